# baseline (speedup 1.0000x reference)
import functools

import jax
import jax.numpy as jnp
from jax.experimental import pallas as pl
from jax.experimental.pallas import tpu as pltpu

_N_SUPPORTS = 2
_MAX_STEP = 2
_BN = 512


def _gc_body(n_batch, d_in, s_ref, x0_ref, w0_ref, wp_ref, b_ref, out_ref,
             xcur_ref):
    p = pl.program_id(0)
    nb = pl.program_id(1)
    rows = pl.ds(nb * _BN, _BN)

    s_blk = s_ref[0]

    def _proj(xb, w_ref2, accumulate):
        for b in range(n_batch):
            contrib = jnp.dot(xb[:, b * d_in:(b + 1) * d_in], w_ref2[0],
                              preferred_element_type=jnp.float32)
            if accumulate:
                out_ref[b, rows, :] += contrib
            else:
                out_ref[b, rows, :] = contrib + b_ref[0, :][None, :]

    @pl.when(p == 0)
    def _init():
        _proj(x0_ref[rows, :], w0_ref, accumulate=False)

    @pl.when(p % 2 == 0)
    def _step1():
        y = jnp.dot(s_blk, x0_ref[...], preferred_element_type=jnp.float32)
        xcur_ref[rows, :] = y
        _proj(y, wp_ref, accumulate=True)

    @pl.when(p % 2 == 1)
    def _step2():
        y = jnp.dot(s_blk, xcur_ref[...], preferred_element_type=jnp.float32)
        x2 = 2.0 * y - x0_ref[rows, :]
        _proj(x2, wp_ref, accumulate=True)


@jax.jit
def kernel(inputs, supports, W, b):
    B, N, D = inputs.shape
    OUT = W.shape[1]
    M = _N_SUPPORTS * _MAX_STEP + 1

    x0 = jnp.transpose(inputs, (1, 0, 2)).reshape(N, B * D)
    w_m = jnp.transpose(W.reshape(D, M, OUT), (1, 0, 2))
    b2 = b.reshape(1, OUT)

    n_phases = _N_SUPPORTS * _MAX_STEP
    grid = (n_phases, N // _BN)

    out = pl.pallas_call(
        functools.partial(_gc_body, B, D),
        grid=grid,
        in_specs=[
            pl.BlockSpec((1, _BN, N), lambda p, nb: (p // 2, nb, 0)),
            pl.BlockSpec((N, B * D), lambda p, nb: (0, 0)),
            pl.BlockSpec((1, D, OUT), lambda p, nb: (0, 0, 0)),
            pl.BlockSpec((1, D, OUT), lambda p, nb: (p + 1, 0, 0)),
            pl.BlockSpec((1, OUT), lambda p, nb: (0, 0)),
        ],
        out_specs=pl.BlockSpec((B, N, OUT), lambda p, nb: (0, 0, 0)),
        out_shape=jax.ShapeDtypeStruct((B, N, OUT), jnp.float32),
        scratch_shapes=[pltpu.VMEM((N, B * D), jnp.float32)],
    )(supports, x0, w_m, w_m, b2)
    return out


# speedup vs baseline: 1.1226x; 1.1226x over previous
import functools

import jax
import jax.numpy as jnp
from jax.experimental import pallas as pl
from jax.experimental.pallas import tpu as pltpu

_N_SUPPORTS = 2
_MAX_STEP = 2
_BN = 1024


def _gc_body(n_batch, d_in, s_ref, x0_ref, w0_ref, wc_ref, b_ref, out_ref,
             xcur_ref):
    p = pl.program_id(0)
    nb = pl.program_id(1)
    rows = pl.ds(nb * _BN, _BN)

    s_blk = s_ref[0].astype(jnp.bfloat16)

    @pl.when(p == 0)
    def _init():
        x0_blk = x0_ref[rows, :]
        for b in range(n_batch):
            acc = jnp.dot(x0_blk[:, b * d_in:(b + 1) * d_in], w0_ref[0],
                          preferred_element_type=jnp.float32)
            out_ref[b, rows, :] = acc + b_ref[0, :][None, :]

    @pl.when(p % 2 == 0)
    def _step1():
        y = jnp.dot(s_blk, x0_ref[...], preferred_element_type=jnp.float32)
        xcur_ref[rows, :] = y.astype(jnp.bfloat16)

    @pl.when(p % 2 == 1)
    def _step2():
        y = jnp.dot(s_blk, xcur_ref[...], preferred_element_type=jnp.float32)
        x2 = (2.0 * y - x0_ref[rows, :].astype(jnp.float32)
              ).astype(jnp.bfloat16)
        x1 = xcur_ref[rows, :]
        for b in range(n_batch):
            cat = jnp.concatenate(
                [x1[:, b * d_in:(b + 1) * d_in],
                 x2[:, b * d_in:(b + 1) * d_in]], axis=1)
            out_ref[b, rows, :] += jnp.dot(cat, wc_ref[0],
                                           preferred_element_type=jnp.float32)


@jax.jit
def kernel(inputs, supports, W, b):
    B, N, D = inputs.shape
    OUT = W.shape[1]
    M = _N_SUPPORTS * _MAX_STEP + 1

    x0 = jnp.transpose(inputs, (1, 0, 2)).reshape(N, B * D)
    x0 = x0.astype(jnp.bfloat16)
    w_m = jnp.transpose(W.reshape(D, M, OUT), (1, 0, 2)).astype(jnp.bfloat16)
    w_cat = w_m[1:].reshape(_N_SUPPORTS, 2 * D, OUT)
    b2 = b.reshape(1, OUT)

    n_phases = _N_SUPPORTS * _MAX_STEP
    grid = (n_phases, N // _BN)

    out = pl.pallas_call(
        functools.partial(_gc_body, B, D),
        grid=grid,
        in_specs=[
            pl.BlockSpec((1, _BN, N), lambda p, nb: (p // 2, nb, 0)),
            pl.BlockSpec((N, B * D), lambda p, nb: (0, 0)),
            pl.BlockSpec((1, D, OUT), lambda p, nb: (0, 0, 0)),
            pl.BlockSpec((1, 2 * D, OUT), lambda p, nb: (p // 2, 0, 0)),
            pl.BlockSpec((1, OUT), lambda p, nb: (0, 0)),
        ],
        out_specs=pl.BlockSpec((B, N, OUT), lambda p, nb: (0, 0, 0)),
        out_shape=jax.ShapeDtypeStruct((B, N, OUT), jnp.float32),
        scratch_shapes=[pltpu.VMEM((N, B * D), jnp.bfloat16)],
    )(supports, x0, w_m, w_cat, b2)
    return out


# speedup vs baseline: 1.2386x; 1.1034x over previous
import functools

import jax
import jax.numpy as jnp
from jax.experimental import pallas as pl
from jax.experimental.pallas import tpu as pltpu

_N_SUPPORTS = 2
_MAX_STEP = 2
_BN = 1024


def _gc_body(n_batch, d_in, in_ref, s_ref, w0_ref, wc_ref, b_ref, out_ref,
             x0_ref, xcur_ref):
    p = pl.program_id(0)
    nb = pl.program_id(1)
    rows = pl.ds(nb * _BN, _BN)

    @pl.when(p == 0)
    def _prep():
        for b in range(n_batch):
            xb = in_ref[b].astype(jnp.bfloat16)
            x0_ref[rows, b * d_in:(b + 1) * d_in] = xb
            acc = jnp.dot(xb, w0_ref[0], preferred_element_type=jnp.float32)
            out_ref[b, rows, :] = acc + b_ref[0, :][None, :]

    @pl.when(p % 2 == 1)
    def _step1():
        s_blk = s_ref[0].astype(jnp.bfloat16)
        y = jnp.dot(s_blk, x0_ref[...], preferred_element_type=jnp.float32)
        xcur_ref[rows, :] = y.astype(jnp.bfloat16)

    @pl.when((p % 2 == 0) & (p > 0))
    def _step2():
        s_blk = s_ref[0].astype(jnp.bfloat16)
        y = jnp.dot(s_blk, xcur_ref[...], preferred_element_type=jnp.float32)
        x2 = (2.0 * y - x0_ref[rows, :].astype(jnp.float32)
              ).astype(jnp.bfloat16)
        x1 = xcur_ref[rows, :]
        for b in range(n_batch):
            cat = jnp.concatenate(
                [x1[:, b * d_in:(b + 1) * d_in],
                 x2[:, b * d_in:(b + 1) * d_in]], axis=1)
            out_ref[b, rows, :] += jnp.dot(cat, wc_ref[0],
                                           preferred_element_type=jnp.float32)


@jax.jit
def kernel(inputs, supports, W, b):
    B, N, D = inputs.shape
    OUT = W.shape[1]
    M = _N_SUPPORTS * _MAX_STEP + 1
    n_blocks = N // _BN

    w_m = jnp.transpose(W.reshape(D, M, OUT), (1, 0, 2)).astype(jnp.bfloat16)
    w0 = w_m[:1]
    w_cat = w_m[1:].reshape(_N_SUPPORTS, 2 * D, OUT)
    b2 = b.reshape(1, OUT)

    grid = (1 + _N_SUPPORTS * _MAX_STEP, n_blocks)

    def im_inputs(p, nb):
        return (0, jnp.where(p == 0, nb, n_blocks - 1), 0)

    def im_support(p, nb):
        return (jnp.where(p == 0, 0, (p - 1) // 2),
                jnp.where(p == 0, 0, nb), 0)

    def im_wcat(p, nb):
        return (jnp.where(p == 0, 0, (p - 1) // 2), 0, 0)

    out = pl.pallas_call(
        functools.partial(_gc_body, B, D),
        grid=grid,
        in_specs=[
            pl.BlockSpec((B, _BN, D), im_inputs),
            pl.BlockSpec((1, _BN, N), im_support),
            pl.BlockSpec((1, D, OUT), lambda p, nb: (0, 0, 0)),
            pl.BlockSpec((1, 2 * D, OUT), im_wcat),
            pl.BlockSpec((1, OUT), lambda p, nb: (0, 0)),
        ],
        out_specs=pl.BlockSpec((B, N, OUT), lambda p, nb: (0, 0, 0)),
        out_shape=jax.ShapeDtypeStruct((B, N, OUT), jnp.float32),
        scratch_shapes=[pltpu.VMEM((N, B * D), jnp.bfloat16),
                        pltpu.VMEM((N, B * D), jnp.bfloat16)],
    )(inputs, supports, w0, w_cat, b2)
    return out


# speedup vs baseline: 1.2412x; 1.0021x over previous
import functools

import jax
import jax.numpy as jnp
from jax.experimental import pallas as pl
from jax.experimental.pallas import tpu as pltpu

_N_SUPPORTS = 2
_MAX_STEP = 2
_BN = 1024


def _gc_body(n_batch, d_in, in_ref, s_ref, w0_ref, wc_ref, b_ref, out_ref,
             x0_ref, xcur_ref):
    p = pl.program_id(0)
    nb = pl.program_id(1)
    rows = pl.ds(nb * _BN, _BN)

    @pl.when(p == 0)
    def _prep():
        for b in range(n_batch):
            xb = in_ref[b].astype(jnp.bfloat16)
            x0_ref[rows, b * d_in:(b + 1) * d_in] = xb
            acc = jnp.dot(xb, w0_ref[0], preferred_element_type=jnp.float32)
            out_ref[b, rows, :] = acc + b_ref[0, :][None, :]

    @pl.when(p % 2 == 1)
    def _step1():
        s_blk = s_ref[0].astype(jnp.bfloat16)
        y = jnp.dot(s_blk, x0_ref[...], preferred_element_type=jnp.float32)
        xcur_ref[rows, :] = y.astype(jnp.bfloat16)

    @pl.when((p % 2 == 0) & (p > 0))
    def _step2():
        s_blk = s_ref[0].astype(jnp.bfloat16)
        y = jnp.dot(s_blk, xcur_ref[...], preferred_element_type=jnp.float32)
        yb = y.astype(jnp.bfloat16)
        x1 = xcur_ref[rows, :]
        for b in range(n_batch):
            cat = jnp.concatenate(
                [x1[:, b * d_in:(b + 1) * d_in],
                 yb[:, b * d_in:(b + 1) * d_in]], axis=1)
            out_ref[b, rows, :] += jnp.dot(cat, wc_ref[0],
                                           preferred_element_type=jnp.float32)


@jax.jit
def kernel(inputs, supports, W, b):
    B, N, D = inputs.shape
    OUT = W.shape[1]
    M = _N_SUPPORTS * _MAX_STEP + 1
    n_blocks = N // _BN

    w_m = jnp.transpose(W.reshape(D, M, OUT), (1, 0, 2))
    w0 = (w_m[0] - w_m[2] - w_m[4]).astype(jnp.bfloat16)[None]
    w_cat = jnp.stack([
        jnp.concatenate([w_m[1 + 2 * i], 2.0 * w_m[2 + 2 * i]], axis=0)
        for i in range(_N_SUPPORTS)]).astype(jnp.bfloat16)
    b2 = b.reshape(1, OUT)

    grid = (1 + _N_SUPPORTS * _MAX_STEP, n_blocks)

    def im_inputs(p, nb):
        return (0, jnp.where(p == 0, nb, n_blocks - 1), 0)

    def im_support(p, nb):
        return (jnp.where(p == 0, 0, (p - 1) // 2),
                jnp.where(p == 0, 0, nb), 0)

    def im_wcat(p, nb):
        return (jnp.where(p == 0, 0, (p - 1) // 2), 0, 0)

    out = pl.pallas_call(
        functools.partial(_gc_body, B, D),
        grid=grid,
        in_specs=[
            pl.BlockSpec((B, _BN, D), im_inputs),
            pl.BlockSpec((1, _BN, N), im_support),
            pl.BlockSpec((1, D, OUT), lambda p, nb: (0, 0, 0)),
            pl.BlockSpec((1, 2 * D, OUT), im_wcat),
            pl.BlockSpec((1, OUT), lambda p, nb: (0, 0)),
        ],
        out_specs=pl.BlockSpec((B, N, OUT), lambda p, nb: (0, 0, 0)),
        out_shape=jax.ShapeDtypeStruct((B, N, OUT), jnp.float32),
        scratch_shapes=[pltpu.VMEM((N, B * D), jnp.bfloat16),
                        pltpu.VMEM((N, B * D), jnp.bfloat16)],
    )(inputs, supports, w0, w_cat, b2)
    return out


# speedup vs baseline: 1.2450x; 1.0031x over previous
import functools

import jax
import jax.numpy as jnp
from jax.experimental import pallas as pl
from jax.experimental.pallas import tpu as pltpu

_N_SUPPORTS = 2
_MAX_STEP = 2
_BN = 1024


def _gc_body(n_batch, d_in, in_ref, s_ref, w0_ref, wc_ref, b_ref, out_ref,
             x0_ref, xcur_ref, acc_ref):
    p = pl.program_id(0)
    nb = pl.program_id(1)
    rows = pl.ds(nb * _BN, _BN)

    @pl.when(p == 0)
    def _prep():
        for b in range(n_batch):
            xb = in_ref[b].astype(jnp.bfloat16)
            x0_ref[rows, b * d_in:(b + 1) * d_in] = xb
            acc = jnp.dot(xb, w0_ref[0], preferred_element_type=jnp.float32)
            acc_ref[b, rows, :] = acc + b_ref[0, :][None, :]

    @pl.when(p % 2 == 1)
    def _step1():
        s_blk = s_ref[0].astype(jnp.bfloat16)
        y = jnp.dot(s_blk, x0_ref[...], preferred_element_type=jnp.float32)
        xcur_ref[rows, :] = y.astype(jnp.bfloat16)

    @pl.when((p % 2 == 0) & (p > 0))
    def _step2():
        s_blk = s_ref[0].astype(jnp.bfloat16)
        y = jnp.dot(s_blk, xcur_ref[...], preferred_element_type=jnp.float32)
        yb = y.astype(jnp.bfloat16)
        x1 = xcur_ref[rows, :]
        for b in range(n_batch):
            cat = jnp.concatenate(
                [x1[:, b * d_in:(b + 1) * d_in],
                 yb[:, b * d_in:(b + 1) * d_in]], axis=1)
            acc_ref[b, rows, :] += jnp.dot(cat, wc_ref[0],
                                           preferred_element_type=jnp.float32)

    @pl.when(p == _N_SUPPORTS * _MAX_STEP)
    def _flush():
        for b in range(n_batch):
            out_ref[b, :, :] = acc_ref[b, rows, :]


@jax.jit
def kernel(inputs, supports, W, b):
    B, N, D = inputs.shape
    OUT = W.shape[1]
    M = _N_SUPPORTS * _MAX_STEP + 1
    n_blocks = N // _BN

    w_m = jnp.transpose(W.reshape(D, M, OUT), (1, 0, 2))
    w0 = (w_m[0] - w_m[2] - w_m[4]).astype(jnp.bfloat16)[None]
    w_cat = jnp.stack([
        jnp.concatenate([w_m[1 + 2 * i], 2.0 * w_m[2 + 2 * i]], axis=0)
        for i in range(_N_SUPPORTS)]).astype(jnp.bfloat16)
    b2 = b.reshape(1, OUT)

    grid = (1 + _N_SUPPORTS * _MAX_STEP, n_blocks)

    def im_inputs(p, nb):
        return (0, jnp.where(p == 0, nb, n_blocks - 1), 0)

    def im_support(p, nb):
        return (jnp.where(p == 0, 0, (p - 1) // 2),
                jnp.where(p == 0, 0, nb), 0)

    def im_wcat(p, nb):
        return (jnp.where(p == 0, 0, (p - 1) // 2), 0, 0)

    out = pl.pallas_call(
        functools.partial(_gc_body, B, D),
        grid=grid,
        in_specs=[
            pl.BlockSpec((B, _BN, D), im_inputs),
            pl.BlockSpec((1, _BN, N), im_support),
            pl.BlockSpec((1, D, OUT), lambda p, nb: (0, 0, 0)),
            pl.BlockSpec((1, 2 * D, OUT), im_wcat),
            pl.BlockSpec((1, OUT), lambda p, nb: (0, 0)),
        ],
        out_specs=pl.BlockSpec(
            (B, _BN, OUT),
            lambda p, nb: (0, jnp.where(p == _N_SUPPORTS * _MAX_STEP, nb, 0),
                           0)),
        out_shape=jax.ShapeDtypeStruct((B, N, OUT), jnp.float32),
        scratch_shapes=[pltpu.VMEM((N, B * D), jnp.bfloat16),
                        pltpu.VMEM((N, B * D), jnp.bfloat16),
                        pltpu.VMEM((B, N, OUT), jnp.float32)],
    )(inputs, supports, w0, w_cat, b2)
    return out
